# baseline (speedup 1.0000x reference)
import functools

import jax
import jax.numpy as jnp
from jax import lax
from jax.experimental import pallas as pl
from jax.experimental.pallas import tpu as pltpu
from jax.experimental.pallas import tpu_sc as plsc


def _mm(a, b):
    return jax.lax.dot_general(a, b, (((1,), (0,)), ((), ())),
                               preferred_element_type=jnp.float32)


def _proj_msg_body(x_ref, winT_ref, bin_ref, wmsgT_ref, bmsg_ref,
                   state_ref, msg_ref):
    st = jnp.maximum(_mm(x_ref[...], winT_ref[...]) + bin_ref[...], 0.0)
    state_ref[...] = st
    msg_ref[...] = jnp.maximum(_mm(st, wmsgT_ref[...]) + bmsg_ref[...], 0.0)


def _gru_core(h_dim, agg_ref, state_ref, wihT_ref, whhT_ref, bih_ref,
              bhh_ref):
    gi = _mm(agg_ref[...], wihT_ref[...]) + bih_ref[...]
    gh = _mm(state_ref[...], whhT_ref[...]) + bhh_ref[...]
    i_r, i_z, i_n = (gi[:, :h_dim], gi[:, h_dim:2 * h_dim], gi[:, 2 * h_dim:])
    h_r, h_z, h_n = (gh[:, :h_dim], gh[:, h_dim:2 * h_dim], gh[:, 2 * h_dim:])
    r = jax.nn.sigmoid(i_r + h_r)
    z = jax.nn.sigmoid(i_z + h_z)
    n = jnp.tanh(i_n + r * h_n)
    return (1.0 - z) * n + z * state_ref[...]


def _gru_msg_body(h_dim, agg_ref, state_ref, wihT_ref, whhT_ref,
                  bih_ref, bhh_ref, wmsgT_ref, bmsg_ref,
                  newstate_ref, msg_ref):
    h = _gru_core(h_dim, agg_ref, state_ref, wihT_ref, whhT_ref,
                  bih_ref, bhh_ref)
    newstate_ref[...] = h
    msg_ref[...] = jnp.maximum(_mm(h, wmsgT_ref[...]) + bmsg_ref[...], 0.0)


def _gru_out_body(h_dim, l_dim, agg_ref, state_ref, wihT_ref, whhT_ref,
                  bih_ref, bhh_ref, woutT_ref, bout_ref, mu_ref, lv_ref):
    h = _gru_core(h_dim, agg_ref, state_ref, wihT_ref, whhT_ref,
                  bih_ref, bhh_ref)
    out = _mm(h, woutT_ref[...]) + bout_ref[...]
    mu_ref[...] = out[:, :l_dim]
    lv_ref[...] = out[:, l_dim:]


def _row_spec(bn, width):
    return pl.BlockSpec((bn, width), lambda i: (i, 0))


def _full_spec(shape):
    nd = len(shape)
    return pl.BlockSpec(shape, lambda i: (0,) * nd)


def _chunk_spec(nch, bn, cw):
    return pl.BlockSpec((nch, bn, cw), lambda i: (0, i, 0))


NQ = 8
NB_ROWS = 1280
ACC_ROWS = 1408
BCAP = 22528
K_B = 128


def _make_sc_scatter(n_nodes, h_dim, num_cores, num_subcores):
    zrows = ACC_ROWS // num_subcores
    wrows = NB_ROWS // num_subcores
    sps = BCAP // num_subcores
    nbatch = sps // K_B
    bpc = NQ // num_cores
    n_out = NQ * NB_ROWS
    mesh = plsc.VectorSubcoreMesh(core_axis_name="c", subcore_axis_name="s")

    nch = h_dim // 128
    spsx = sps * nch
    drows = nbatch * nch
    drows_p = (drows + 7) // 8 * 8
    zrowsx = ACC_ROWS * nch // num_subcores
    wrowsx = NB_ROWS * nch // num_subcores

    @functools.partial(
        pl.kernel,
        out_type=jax.ShapeDtypeStruct((n_out * nch, 128), jnp.float32),
        mesh=mesh,
        scratch_types=[
            pltpu.VMEM((spsx,), jnp.int32),
            pltpu.VMEM((drows_p, K_B), jnp.int32),
            pltpu.VMEM((K_B * nch, 128), jnp.float32),
            pltpu.VMEM_SHARED((ACC_ROWS * nch, 128), jnp.float32),
            pltpu.SemaphoreType.DMA,
        ],
    )
    def sc_scatter(msg_hbm, psrc_hbm, pdst4_hbm, zeros_hbm, out_hbm,
                   src_v, dst_v, rows_v, acc_sh, gsem):
        core = lax.axis_index("c")
        sub = lax.axis_index("s")

        for p in range(bpc):
            q = core * bpc + p

            pltpu.sync_copy(zeros_hbm.at[pl.ds(sub * zrowsx, zrowsx)],
                            acc_sh.at[pl.ds(sub * zrowsx, zrowsx)])
            pltpu.sync_copy(psrc_hbm.at[q].at[pl.ds(sub * spsx, spsx)],
                            src_v)
            pltpu.sync_copy(pdst4_hbm.at[q, sub], dst_v)
            plsc.subcore_barrier()

            for i in range(nbatch):
                pltpu.async_copy(
                    msg_hbm.at[src_v.at[pl.ds(i * K_B * nch, K_B * nch)]],
                    rows_v, gsem)
                pltpu.make_async_copy(msg_hbm.at[pl.ds(0, K_B * nch)],
                                      rows_v, gsem).wait()
                for j in range(nch):
                    pltpu.sync_copy(rows_v.at[pl.ds(j * K_B, K_B)],
                                    acc_sh.at[dst_v.at[nch * i + j]],
                                    add=True)

            plsc.subcore_barrier()
            pltpu.sync_copy(
                acc_sh.at[pl.ds(sub * wrowsx, wrowsx)],
                out_hbm.at[pl.ds(q * (NB_ROWS * nch) + sub * wrowsx,
                                 wrowsx)])
            plsc.subcore_barrier()

    return sc_scatter


def kernel(x, edge_index, W_in, b_in, W_msg, b_msg, W_ih, W_hh, b_ih, b_hh,
           W_mu, b_mu, W_lv, b_lv):
    n, f_dim = x.shape
    h_dim = W_in.shape[0]
    n_rounds = W_msg.shape[0]
    l_dim = W_mu.shape[0]
    e = edge_index.shape[1]

    bn = 1000 if n % 1000 == 0 else n
    grid = (n // bn,)
    num_cores, num_subcores = 2, 16

    src = edge_index[0].astype(jnp.int32)
    dst = edge_index[1].astype(jnp.int32)
    b = jnp.minimum(dst // NB_ROWS, NQ - 1)
    dloc = jnp.minimum(dst - b * NB_ROWS, NB_ROWS)
    order = jnp.argsort(b, stable=True)
    b_s = b[order]
    counts = jnp.zeros((NQ,), jnp.int32).at[b].add(1)
    starts = jnp.cumsum(counts) - counts
    rank = jnp.arange(e, dtype=jnp.int32) - starts[b_s]
    rank = jnp.minimum(rank, BCAP - 1)
    nch = h_dim // 128
    sps = BCAP // num_subcores
    spsx = sps * nch
    nbatch = sps // K_B
    drows_p = (nbatch * nch + 7) // 8 * 8
    s_ = rank // sps
    i_ = (rank % sps) // K_B
    l_ = rank % K_B
    off = jnp.arange(nch, dtype=jnp.int32)[None, :]
    srcx = (src[order] * nch)[:, None] + off
    valx = (dloc[order] * nch)[:, None] + off
    pos_sx = ((b_s * num_subcores + s_) * spsx
              + i_ * K_B * nch + l_ * nch)[:, None] + off
    psrc = jnp.zeros((NQ * num_subcores * spsx,), jnp.int32) \
        .at[pos_sx.ravel()].set(srcx.ravel())
    psrc3 = psrc.reshape(NQ, num_subcores * spsx)
    pp = (l_ * nch)[:, None] + off
    rowx = (i_ * nch)[:, None] + pp // K_B
    lanex = pp % K_B
    pos_dx = (((b_s * num_subcores + s_) * drows_p)[:, None] + rowx) * K_B \
        + lanex
    pdst = jnp.full((NQ * num_subcores * drows_p * K_B,), NB_ROWS * nch,
                    jnp.int32).at[pos_dx.ravel()].set(valx.ravel())
    pdst4 = pdst.reshape(NQ, num_subcores, drows_p, K_B)
    zeros = jnp.zeros((ACC_ROWS * nch, 128), jnp.float32)

    winT = W_in.T
    wmsgT = jnp.transpose(W_msg, (0, 2, 1))
    wihT = W_ih.T
    whhT = W_hh.T
    bin2 = b_in.reshape(1, h_dim)
    bmsg2 = b_msg.reshape(n_rounds, 1, h_dim)
    bih2 = b_ih.reshape(1, 3 * h_dim)
    bhh2 = b_hh.reshape(1, 3 * h_dim)
    woutT = jnp.concatenate([W_mu.T, W_lv.T], axis=1)
    bout2 = jnp.concatenate([b_mu, b_lv]).reshape(1, 2 * l_dim)

    f32 = jnp.float32
    state_sds = jax.ShapeDtypeStruct((n, h_dim), f32)
    msg_sds = jax.ShapeDtypeStruct((n, h_dim), f32)

    proj_call = pl.pallas_call(
        _proj_msg_body,
        grid=grid,
        in_specs=[_row_spec(bn, f_dim), _full_spec((f_dim, h_dim)),
                  _full_spec((1, h_dim)), _full_spec((h_dim, h_dim)),
                  _full_spec((1, h_dim))],
        out_specs=[_row_spec(bn, h_dim), _row_spec(bn, h_dim)],
        out_shape=[state_sds, msg_sds],
    )

    gru_msg_call = pl.pallas_call(
        functools.partial(_gru_msg_body, h_dim),
        grid=grid,
        in_specs=[_row_spec(bn, h_dim), _row_spec(bn, h_dim),
                  _full_spec((h_dim, 3 * h_dim)),
                  _full_spec((h_dim, 3 * h_dim)),
                  _full_spec((1, 3 * h_dim)), _full_spec((1, 3 * h_dim)),
                  _full_spec((h_dim, h_dim)), _full_spec((1, h_dim))],
        out_specs=[_row_spec(bn, h_dim), _row_spec(bn, h_dim)],
        out_shape=[state_sds, msg_sds],
    )

    gru_out_call = pl.pallas_call(
        functools.partial(_gru_out_body, h_dim, l_dim),
        grid=grid,
        in_specs=[_row_spec(bn, h_dim), _row_spec(bn, h_dim),
                  _full_spec((h_dim, 3 * h_dim)),
                  _full_spec((h_dim, 3 * h_dim)),
                  _full_spec((1, 3 * h_dim)), _full_spec((1, 3 * h_dim)),
                  _full_spec((h_dim, 2 * l_dim)), _full_spec((1, 2 * l_dim))],
        out_specs=[_row_spec(bn, l_dim), _row_spec(bn, l_dim)],
        out_shape=[jax.ShapeDtypeStruct((n, l_dim), f32),
                   jax.ShapeDtypeStruct((n, l_dim), f32)],
    )

    sc_scatter = _make_sc_scatter(n, h_dim, num_cores, num_subcores)

    state, msg = proj_call(x, winT, bin2, wmsgT[0], bmsg2[0])
    for r in range(n_rounds):
        aggx = sc_scatter(msg.reshape(n * nch, 128), psrc3, pdst4, zeros)
        agg = aggx.reshape(NQ * NB_ROWS, h_dim)
        if r < n_rounds - 1:
            state, msg = gru_msg_call(agg, state, wihT, whhT, bih2, bhh2,
                                      wmsgT[r + 1], bmsg2[r + 1])
        else:
            mu, lv = gru_out_call(agg, state, wihT, whhT, bih2, bhh2,
                                  woutT, bout2)
    return (mu, lv)
